# Initial kernel scaffold; baseline (speedup 1.0000x reference)
import jax
import jax.numpy as jnp
from jax.experimental import pallas as pl


def kernel(feat_0, coord_0, feat_1, coord_1):
    raise NotImplementedError("write your pallas kernel here")


# speedup vs baseline: 12.4507x; 12.4507x over previous
import functools

import jax
import jax.numpy as jnp
from jax.experimental import pallas as pl
from jax.experimental.pallas import tpu as pltpu

_TEMP = 0.1
_K = 4
_BIG = 3.0e38


def _dir_body(ca_ref, fa_ref, cbt_ref, fbt_ref, out_ref):
    ca = ca_ref[...]
    fa = fa_ref[...]
    cbt = cbt_ref[...]
    fbt = fbt_ref[...]

    na2 = jnp.sum(fa * fa, axis=1, keepdims=True)
    fa_n = fa * jax.lax.rsqrt(jnp.maximum(na2, 1e-24))

    nb2 = jnp.sum(fbt * fbt, axis=0, keepdims=True)
    rnb = jax.lax.rsqrt(jnp.maximum(nb2, 1e-24))

    s = jnp.dot(fa_n, fbt, preferred_element_type=jnp.float32) * rnb

    q2 = jnp.sum(ca * ca, axis=1, keepdims=True)
    r2 = jnp.sum(cbt * cbt, axis=0, keepdims=True)
    cross = jnp.dot(ca, cbt, preferred_element_type=jnp.float32)
    d2 = q2 - 2.0 * cross + r2

    sims = []
    for _ in range(_K):
        m = jnp.min(d2, axis=1, keepdims=True)
        sel = d2 == m
        sims.append(jnp.sum(jnp.where(sel, s, 0.0), axis=1, keepdims=True))
        d2 = jnp.where(sel, _BIG, d2)

    logits = [si * (1.0 / _TEMP) for si in sims]
    mx = jnp.maximum(jnp.maximum(logits[0], logits[1]),
                     jnp.maximum(logits[2], logits[3]))
    es = [jnp.exp(li - mx) for li in logits]
    z = es[0] + es[1] + es[2] + es[3]
    p2 = (es[0] * es[0] + es[1] * es[1] + es[2] * es[2] + es[3] * es[3]) / (z * z)
    loss = -jnp.log(p2 + 1e-12)

    @pl.when(pl.program_id(0) == 0)
    def _init():
        out_ref[0, 0] = 0.0

    out_ref[0, 0] += jnp.sum(loss)


def _directional_sums(coord_a, feat_a, coord_b, feat_b, bq):
    na, c = feat_a.shape
    nb = feat_b.shape[0]
    grid = na // bq
    ca = jnp.pad(coord_a, ((0, 0), (0, 5)))
    cbt = jnp.pad(coord_b, ((0, 0), (0, 5))).T
    fbt = feat_b.T
    out = pl.pallas_call(
        _dir_body,
        grid=(grid,),
        in_specs=[
            pl.BlockSpec((bq, 8), lambda i: (i, 0)),
            pl.BlockSpec((bq, c), lambda i: (i, 0)),
            pl.BlockSpec((8, nb), lambda i: (0, 0)),
            pl.BlockSpec((c, nb), lambda i: (0, 0)),
        ],
        out_specs=pl.BlockSpec((1, 1), lambda i: (0, 0),
                               memory_space=pltpu.SMEM),
        out_shape=jax.ShapeDtypeStruct((1, 1), jnp.float32),
        compiler_params=pltpu.CompilerParams(
            dimension_semantics=("arbitrary",)),
    )(ca, feat_a, cbt, fbt)
    return out


def kernel(feat_0, coord_0, feat_1, coord_1):
    n0 = feat_0.shape[0]
    n1 = feat_1.shape[0]
    s01 = _directional_sums(coord_0, feat_0, coord_1, feat_1, bq=256)
    s10 = _directional_sums(coord_1, feat_1, coord_0, feat_0, bq=256)
    loss0 = s01[0, 0] / n0
    loss1 = s10[0, 0] / n1
    return 0.5 * (loss0 + loss1)


# speedup vs baseline: 12.8554x; 1.0325x over previous
import functools

import jax
import jax.numpy as jnp
from jax.experimental import pallas as pl
from jax.experimental.pallas import tpu as pltpu

_TEMP = 0.1
_K = 4
_BIG = 3.0e38


def _dir_body(ca_ref, fa_ref, cbt_ref, fbt_ref, out_ref, fbtn_ref, r2_ref):
    ca = ca_ref[...]
    fa = fa_ref[...]
    cbt = cbt_ref[...]

    @pl.when(pl.program_id(0) == 0)
    def _prep():
        fbt = fbt_ref[...]
        nb2 = jnp.sum(fbt * fbt, axis=0, keepdims=True)
        rnb = jax.lax.rsqrt(jnp.maximum(nb2, 1e-24))
        fbtn_ref[...] = (fbt * rnb).astype(jnp.bfloat16)
        r2_ref[...] = jnp.sum(cbt * cbt, axis=0, keepdims=True)

    na2 = jnp.sum(fa * fa, axis=1, keepdims=True)
    fa_n = (fa * jax.lax.rsqrt(jnp.maximum(na2, 1e-24))).astype(jnp.bfloat16)

    s = jnp.dot(fa_n, fbtn_ref[...], preferred_element_type=jnp.float32)

    q2 = jnp.sum(ca * ca, axis=1, keepdims=True)
    cross = jnp.dot(ca, cbt, preferred_element_type=jnp.float32)
    d2 = q2 - 2.0 * cross + r2_ref[...]

    sims = []
    for _ in range(_K):
        m = jnp.min(d2, axis=1, keepdims=True)
        sel = d2 == m
        sims.append(jnp.sum(jnp.where(sel, s, 0.0), axis=1, keepdims=True))
        d2 = jnp.where(sel, _BIG, d2)

    logits = [si * (1.0 / _TEMP) for si in sims]
    mx = jnp.maximum(jnp.maximum(logits[0], logits[1]),
                     jnp.maximum(logits[2], logits[3]))
    es = [jnp.exp(li - mx) for li in logits]
    z = es[0] + es[1] + es[2] + es[3]
    p2 = (es[0] * es[0] + es[1] * es[1] + es[2] * es[2] + es[3] * es[3]) / (z * z)
    loss = -jnp.log(p2 + 1e-12)

    @pl.when(pl.program_id(0) == 0)
    def _init():
        out_ref[0, 0] = 0.0

    out_ref[0, 0] += jnp.sum(loss)


def _directional_sums(coord_a, feat_a, coord_b, feat_b, bq):
    na, c = feat_a.shape
    nb = feat_b.shape[0]
    grid = na // bq
    ca = jnp.pad(coord_a, ((0, 0), (0, 5)))
    cbt = jnp.pad(coord_b, ((0, 0), (0, 5))).T
    fbt = feat_b.T
    out = pl.pallas_call(
        _dir_body,
        grid=(grid,),
        in_specs=[
            pl.BlockSpec((bq, 8), lambda i: (i, 0)),
            pl.BlockSpec((bq, c), lambda i: (i, 0)),
            pl.BlockSpec((8, nb), lambda i: (0, 0)),
            pl.BlockSpec((c, nb), lambda i: (0, 0)),
        ],
        out_specs=pl.BlockSpec((1, 1), lambda i: (0, 0),
                               memory_space=pltpu.SMEM),
        out_shape=jax.ShapeDtypeStruct((1, 1), jnp.float32),
        scratch_shapes=[
            pltpu.VMEM((c, nb), jnp.bfloat16),
            pltpu.VMEM((1, nb), jnp.float32),
        ],
        compiler_params=pltpu.CompilerParams(
            dimension_semantics=("arbitrary",)),
    )(ca, feat_a, cbt, fbt)
    return out


def kernel(feat_0, coord_0, feat_1, coord_1):
    n0 = feat_0.shape[0]
    n1 = feat_1.shape[0]
    s01 = _directional_sums(coord_0, feat_0, coord_1, feat_1, bq=256)
    s10 = _directional_sums(coord_1, feat_1, coord_0, feat_0, bq=256)
    loss0 = s01[0, 0] / n0
    loss1 = s10[0, 0] / n1
    return 0.5 * (loss0 + loss1)
